# baseline (speedup 1.0000x reference)
import jax
import jax.numpy as jnp
from jax import lax
from jax.experimental import pallas as pl
from jax.experimental.pallas import tpu as pltpu
from jax.experimental.pallas import tpu_sc as plsc

_BATCH = 16384
_IN_COLS = 40
_OUT_COLS = 69
_NUM_WORKERS = 32
_ROWS_PER = _BATCH // _NUM_WORKERS
_CHUNK = 256


def _sc_body(in_hbm, w0_hbm, w12_hbm, out_hbm, in_v, out_v, w0_v, w12_v):
    wid = lax.axis_index("s") * 2 + lax.axis_index("c")
    pltpu.sync_copy(w0_hbm, w0_v)
    pltpu.sync_copy(w12_hbm, w12_v)

    for chunk in range(_ROWS_PER // _CHUNK):
        base = wid * _ROWS_PER + chunk * _CHUNK
        pltpu.sync_copy(in_hbm.at[pl.ds(base * _IN_COLS, _CHUNK * _IN_COLS)],
                        in_v)
        out_v[pl.ds(0, 16)] = in_v[pl.ds(3, 16)]
        pltpu.sync_copy(out_v,
                        out_hbm.at[pl.ds(base * _OUT_COLS,
                                         _CHUNK * _OUT_COLS)])


def kernel(inputs, W0, W1, W2):
    mesh = plsc.VectorSubcoreMesh(core_axis_name="c", subcore_axis_name="s")
    fn = pl.kernel(
        _sc_body,
        out_type=jax.ShapeDtypeStruct((_BATCH * _OUT_COLS,), jnp.float32),
        mesh=mesh,
        scratch_types=[
            pltpu.VMEM((_CHUNK * _IN_COLS,), jnp.float32),
            pltpu.VMEM((_CHUNK * _OUT_COLS,), jnp.float32),
            pltpu.VMEM((65 * 16,), jnp.float32),
            pltpu.VMEM((33 * 8 + 17 * 8,), jnp.float32),
        ],
        compiler_params=pltpu.CompilerParams(needs_layout_passes=False),
    )
    w12 = jnp.concatenate([W1.reshape(-1), W2.reshape(-1)])
    out = fn(inputs.reshape(-1), W0.reshape(-1), w12)
    return out.reshape(_BATCH, _OUT_COLS)


# speedup vs baseline: 1.7512x; 1.7512x over previous
import jax
import jax.numpy as jnp
from jax.experimental import pallas as pl
from jax.experimental.pallas import tpu as pltpu

_BATCH = 16384
_IN_COLS = 40
_OUT_COLS = 69
_BLK = 1024


def _tc_body(wcat_ref, in_ref, out_ref):
    x = in_ref[...]
    ids = jnp.round(x[:, 0:3]).astype(jnp.int32)
    b0 = jnp.minimum(ids[:, 0:1], 64)
    b1 = jnp.minimum(ids[:, 1:2], 32) + 65
    b2 = jnp.minimum(ids[:, 2:3], 16) + 98
    lane = jax.lax.broadcasted_iota(jnp.int32, (_BLK, 128), 1)
    oh = ((lane == b0) | (lane == b1) | (lane == b2)).astype(jnp.float32)
    emb = jnp.dot(oh, wcat_ref[...], preferred_element_type=jnp.float32)
    out_ref[:, 0:37] = x[:, 3:40]
    out_ref[:, 37:69] = emb


def kernel(inputs, W0, W1, W2):
    wcat = jnp.zeros((128, 32), jnp.float32)
    wcat = wcat.at[0:65, 0:16].set(W0)
    wcat = wcat.at[65:98, 16:24].set(W1)
    wcat = wcat.at[98:115, 24:32].set(W2)
    return pl.pallas_call(
        _tc_body,
        grid=(_BATCH // _BLK,),
        in_specs=[
            pl.BlockSpec((128, 32), lambda i: (0, 0)),
            pl.BlockSpec((_BLK, _IN_COLS), lambda i: (i, 0)),
        ],
        out_specs=pl.BlockSpec((_BLK, _OUT_COLS), lambda i: (i, 0)),
        out_shape=jax.ShapeDtypeStruct((_BATCH, _OUT_COLS), jnp.float32),
        compiler_params=pltpu.CompilerParams(
            dimension_semantics=("arbitrary",)),
    )(wcat, inputs)


# speedup vs baseline: 1.9587x; 1.1185x over previous
import jax
import jax.numpy as jnp
from jax.experimental import pallas as pl
from jax.experimental.pallas import tpu as pltpu

_BATCH = 16384
_IN_COLS = 40
_OUT_COLS = 69
_BLK = 4096


def _tc_body(wcat_ref, in_ref, out_ref):
    x = in_ref[...]
    ids = jnp.round(x[:, 0:3]).astype(jnp.int32)
    b0 = jnp.minimum(ids[:, 0:1], 64)
    b1 = jnp.minimum(ids[:, 1:2], 32) + 65
    b2 = jnp.minimum(ids[:, 2:3], 16) + 98
    lane = jax.lax.broadcasted_iota(jnp.int32, (_BLK, 128), 1)
    oh = ((lane == b0) | (lane == b1) | (lane == b2)).astype(jnp.float32)
    emb = jnp.dot(oh, wcat_ref[...], preferred_element_type=jnp.float32)
    out_ref[:, 0:37] = x[:, 3:40]
    out_ref[:, 37:69] = emb


def kernel(inputs, W0, W1, W2):
    wcat = jnp.zeros((128, 32), jnp.float32)
    wcat = wcat.at[0:65, 0:16].set(W0)
    wcat = wcat.at[65:98, 16:24].set(W1)
    wcat = wcat.at[98:115, 24:32].set(W2)
    return pl.pallas_call(
        _tc_body,
        grid=(_BATCH // _BLK,),
        in_specs=[
            pl.BlockSpec((128, 32), lambda i: (0, 0)),
            pl.BlockSpec((_BLK, _IN_COLS), lambda i: (i, 0)),
        ],
        out_specs=pl.BlockSpec((_BLK, _OUT_COLS), lambda i: (i, 0)),
        out_shape=jax.ShapeDtypeStruct((_BATCH, _OUT_COLS), jnp.float32),
        compiler_params=pltpu.CompilerParams(
            dimension_semantics=("arbitrary",)),
    )(wcat, inputs)


# speedup vs baseline: 4.9191x; 2.5114x over previous
import jax
import jax.numpy as jnp
from jax.experimental import pallas as pl
from jax.experimental.pallas import tpu as pltpu

_BATCH = 16384
_IN_COLS = 40
_OUT_COLS = 69
_BLKC = 2048


def _tc_body(wcat_ref, in_ref, out_ref):
    x = in_ref[...]
    ids = jnp.round(x[0:3, :]).astype(jnp.int32)
    b0 = jnp.minimum(ids[0:1, :], 64)
    b1 = jnp.minimum(ids[1:2, :], 32) + 65
    b2 = jnp.minimum(ids[2:3, :], 16) + 98
    sub = jax.lax.broadcasted_iota(jnp.int32, (128, _BLKC), 0)
    oh = ((sub == b0) | (sub == b1) | (sub == b2)).astype(jnp.float32)
    emb = jnp.dot(wcat_ref[...], oh, preferred_element_type=jnp.float32)
    out_ref[0:37, :] = x[3:40, :]
    out_ref[37:69, :] = emb


def kernel(inputs, W0, W1, W2):
    wcat = jnp.zeros((32, 128), jnp.float32)
    wcat = wcat.at[0:16, 0:65].set(W0.T)
    wcat = wcat.at[16:24, 65:98].set(W1.T)
    wcat = wcat.at[24:32, 98:115].set(W2.T)
    outT = pl.pallas_call(
        _tc_body,
        grid=(_BATCH // _BLKC,),
        in_specs=[
            pl.BlockSpec((32, 128), lambda i: (0, 0)),
            pl.BlockSpec((_IN_COLS, _BLKC), lambda i: (0, i)),
        ],
        out_specs=pl.BlockSpec((_OUT_COLS, _BLKC), lambda i: (0, i)),
        out_shape=jax.ShapeDtypeStruct((_OUT_COLS, _BATCH), jnp.float32),
        compiler_params=pltpu.CompilerParams(
            dimension_semantics=("arbitrary",)),
    )(wcat, inputs.T)
    return outT.T


# speedup vs baseline: 5.6750x; 1.1537x over previous
import jax
import jax.numpy as jnp
from jax.experimental import pallas as pl
from jax.experimental.pallas import tpu as pltpu

_BATCH = 16384
_IN_COLS = 40
_OUT_COLS = 69
_BLKC = 4096


def _tc_body(wcat_ref, in_ref, out_ref):
    x = in_ref[...]
    ids = jnp.round(x[0:3, :]).astype(jnp.int32)
    b0 = jnp.minimum(ids[0:1, :], 64)
    b1 = jnp.minimum(ids[1:2, :], 32) + 65
    b2 = jnp.minimum(ids[2:3, :], 16) + 98
    sub = jax.lax.broadcasted_iota(jnp.int32, (128, _BLKC), 0)
    oh = ((sub == b0) | (sub == b1) | (sub == b2)).astype(jnp.float32)
    emb = jnp.dot(wcat_ref[...], oh, preferred_element_type=jnp.float32)
    out_ref[0:37, :] = x[3:40, :]
    out_ref[37:69, :] = emb


def kernel(inputs, W0, W1, W2):
    wcat = jnp.zeros((32, 128), jnp.float32)
    wcat = wcat.at[0:16, 0:65].set(W0.T)
    wcat = wcat.at[16:24, 65:98].set(W1.T)
    wcat = wcat.at[24:32, 98:115].set(W2.T)
    outT = pl.pallas_call(
        _tc_body,
        grid=(_BATCH // _BLKC,),
        in_specs=[
            pl.BlockSpec((32, 128), lambda i: (0, 0)),
            pl.BlockSpec((_IN_COLS, _BLKC), lambda i: (0, i)),
        ],
        out_specs=pl.BlockSpec((_OUT_COLS, _BLKC), lambda i: (0, i)),
        out_shape=jax.ShapeDtypeStruct((_OUT_COLS, _BATCH), jnp.float32),
        compiler_params=pltpu.CompilerParams(
            dimension_semantics=("arbitrary",)),
    )(wcat, inputs.T)
    return outT.T


# speedup vs baseline: 5.8450x; 1.0300x over previous
import jax
import jax.numpy as jnp
from jax.experimental import pallas as pl
from jax.experimental.pallas import tpu as pltpu

_BATCH = 16384
_IN_COLS = 40
_OUT_COLS = 69
_BLKC = 8192


def _tc_body(wcat_ref, in_ref, out_ref):
    x = in_ref[...]
    ids = jnp.round(x[0:3, :]).astype(jnp.int32)
    b0 = jnp.minimum(ids[0:1, :], 64)
    b1 = jnp.minimum(ids[1:2, :], 32) + 65
    b2 = jnp.minimum(ids[2:3, :], 16) + 98
    sub = jax.lax.broadcasted_iota(jnp.int32, (128, _BLKC), 0)
    oh = ((sub == b0) | (sub == b1) | (sub == b2)).astype(jnp.float32)
    emb = jnp.dot(wcat_ref[...], oh, preferred_element_type=jnp.float32)
    out_ref[0:37, :] = x[3:40, :]
    out_ref[37:69, :] = emb


def kernel(inputs, W0, W1, W2):
    wcat = jnp.zeros((32, 128), jnp.float32)
    wcat = wcat.at[0:16, 0:65].set(W0.T)
    wcat = wcat.at[16:24, 65:98].set(W1.T)
    wcat = wcat.at[24:32, 98:115].set(W2.T)
    outT = pl.pallas_call(
        _tc_body,
        grid=(_BATCH // _BLKC,),
        in_specs=[
            pl.BlockSpec((32, 128), lambda i: (0, 0)),
            pl.BlockSpec((_IN_COLS, _BLKC), lambda i: (0, i)),
        ],
        out_specs=pl.BlockSpec((_OUT_COLS, _BLKC), lambda i: (0, i)),
        out_shape=jax.ShapeDtypeStruct((_OUT_COLS, _BATCH), jnp.float32),
        compiler_params=pltpu.CompilerParams(
            dimension_semantics=("arbitrary",)),
    )(wcat, inputs.T)
    return outT.T
